# baseline (speedup 1.0000x reference)
import jax
import jax.numpy as jnp
from jax import lax
from jax.experimental import pallas as pl
from jax.experimental.pallas import tpu as pltpu
from jax.experimental.pallas import tpu_sc as plsc

B = 4
N = 16384
G = 512
M = 32
NR = 128
NC = 128

_BIG = 1 << 30


BPP = 2


def _fps_body(xs_ref, xt_ref, cs_ref):
    fiota = (lax.broadcasted_iota(jnp.int32, (NR, NC), 0) * NC
             + lax.broadcasted_iota(jnp.int32, (NR, NC), 1))
    dists0 = jnp.full((NR, NC), jnp.inf, dtype=jnp.float32)

    init = []
    for b in range(BPP):
        init += [xs_ref[b, 0, 0], xs_ref[b, 1, 0], xs_ref[b, 2, 0], dists0]

    def step(g, carry):
        out = []
        for b in range(BPP):
            px, py, pz, dists = carry[4 * b:4 * b + 4]
            cs_ref[b, 0, g] = px
            cs_ref[b, 1, g] = py
            cs_ref[b, 2, g] = pz
            dx = xt_ref[b, 0] - px
            dy = xt_ref[b, 1] - py
            dz = xt_ref[b, 2] - pz
            d = dx * dx + dy * dy + dz * dz
            dists = jnp.minimum(dists, d)
            m = jnp.max(dists)
            sel = jnp.where(dists == m, fiota, _BIG)
            i = jnp.min(sel)
            out += [xs_ref[b, 0, i], xs_ref[b, 1, i], xs_ref[b, 2, i],
                    dists]
        return tuple(out)

    lax.fori_loop(0, G, step, tuple(init))


def _fps_call(xflat, xt, interpret=False):
    call = pl.pallas_call(
        _fps_body,
        grid=(1,),
        in_specs=[
            pl.BlockSpec((BPP, 3, N), lambda i: (0, 0, 0),
                         memory_space=pltpu.SMEM),
            pl.BlockSpec((BPP, 3, NR, NC), lambda i: (0, 0, 0, 0)),
        ],
        out_specs=pl.BlockSpec((BPP, 3, G), lambda i: (0, 0, 0),
                               memory_space=pltpu.SMEM),
        out_shape=jax.ShapeDtypeStruct((BPP, 3, G), jnp.float32),
        interpret=interpret,
    )
    return jnp.concatenate(
        [call(xflat[p:p + BPP], xt[p:p + BPP])
         for p in range(0, B, BPP)], axis=0)


CPB = 8


def _d2_body(centers_ref, xt_ref, d2_ref, t_ref, rm_ref):
    x = xt_ref[0, 0]
    y = xt_ref[0, 1]
    z = xt_ref[0, 2]
    ident = (lax.broadcasted_iota(jnp.int32, (NR, NR), 0)
             == lax.broadcasted_iota(jnp.int32, (NR, NR), 1)
             ).astype(jnp.float32)
    neg_inf = jnp.float32(-jnp.inf)
    for c in range(CPB):
        cx = centers_ref[0, c, 0]
        cy = centers_ref[0, c, 1]
        cz = centers_ref[0, c, 2]
        dx = cx - x
        dy = cy - y
        dz = cz - z
        d2 = dx * dx + dy * dy + dz * dz
        d2_ref[c] = d2
        rm = jnp.min(d2, axis=1, keepdims=True)
        rmt = lax.dot_general(rm, ident, (((0,), (0,)), ((), ())),
                              precision=lax.Precision.HIGHEST)
        rank = jnp.sum((rmt < rm).astype(jnp.int32), axis=1, keepdims=True)
        t = jnp.max(jnp.where(rank < M, rm, neg_inf))
        t_ref[0, 0, c] = t
        rm_ref[c] = rmt


def _d2_call(centers, xt, interpret=False):
    return pl.pallas_call(
        _d2_body,
        grid=(B, G // CPB),
        in_specs=[
            pl.BlockSpec((1, CPB, 3), lambda b, j: (b, j, 0),
                         memory_space=pltpu.SMEM),
            pl.BlockSpec((1, 3, NR, NC), lambda b, j: (b, 0, 0, 0)),
        ],
        out_specs=[
            pl.BlockSpec((CPB, NR, NC), lambda b, j: (b * (G // CPB) + j, 0, 0)),
            pl.BlockSpec((1, 1, CPB), lambda b, j: (b * (G // CPB) + j, 0, 0),
                         memory_space=pltpu.SMEM),
            pl.BlockSpec((CPB, 1, NR), lambda b, j: (b * (G // CPB) + j, 0, 0)),
        ],
        out_shape=[
            jax.ShapeDtypeStruct((B * G, NR, NC), jnp.float32),
            jax.ShapeDtypeStruct((B * G // CPB, 1, CPB), jnp.float32),
            jax.ShapeDtypeStruct((B * G, 1, NR), jnp.float32),
        ],
        compiler_params=pltpu.CompilerParams(
            dimension_semantics=("parallel", "parallel")),
        interpret=interpret,
    )(centers, xt)


NW = 32
WPB = NW // B
RPW = G // WPB
EPW = RPW * M
CAP = 1024
CAPR = 48


def _sel_body(xt_hbm, ct_hbm, t_hbm, d2_hbm, rm_hbm, out_hbm,
              xv, yv, zv, cxv, cyv, czv, tv, rmv, blist, idxv, gbuf, dv,
              candv, candi, ov, sem):
    w = lax.axis_index("s") * 2 + lax.axis_index("c")
    b = w // WPB
    r = w % WPB
    pltpu.sync_copy(xt_hbm.at[pl.ds(b * 3 * N, N)], xv)
    pltpu.sync_copy(xt_hbm.at[pl.ds((b * 3 + 1) * N, N)], yv)
    pltpu.sync_copy(xt_hbm.at[pl.ds((b * 3 + 2) * N, N)], zv)
    pltpu.sync_copy(ct_hbm.at[pl.ds(b * 3 * G, G)], cxv)
    pltpu.sync_copy(ct_hbm.at[pl.ds((b * 3 + 1) * G, G)], cyv)
    pltpu.sync_copy(ct_hbm.at[pl.ds((b * 3 + 2) * G, G)], czv)
    row0 = b * G + r * RPW
    pltpu.sync_copy(t_hbm.at[pl.ds(row0, RPW)], tv)

    lane = lax.iota(jnp.int32, 16)
    inf16 = jnp.full((16,), jnp.inf, dtype=jnp.float32)
    big16 = jnp.full((16,), _BIG, dtype=jnp.int32)

    def row_body(q, _):
        pltpu.sync_copy(rm_hbm.at[pl.ds((row0 + q) * NR, NR)], rmv)
        tsv = plsc.load_gather(tv, [jnp.full((16,), q, dtype=jnp.int32)])

        def fchunk(t, off):
            rv = rmv[pl.ds(t * 16, 16)]
            mask = rv <= tsv
            plsc.store_compressed(blist.at[pl.ds(off, 16)],
                                  t * 16 + lane, mask=mask)
            return off + jnp.sum(mask.astype(jnp.int32))

        nb = lax.fori_loop(0, NR // 16, fchunk, jnp.int32(0))

        rowbase = (row0 + q) * NR
        for t in range(CAPR // 16):
            c = blist[pl.ds(t * 16, 16)] + rowbase
            m = (t * 16 + lane) < nb
            idxv[pl.ds(t * 16, 16)] = jnp.where(m, c, 0)
        pltpu.async_copy(d2_hbm.at[idxv], gbuf, sem).wait()
        nbc = jnp.minimum(nb, jnp.int32(CAPR))

        def rchunk(u, carry):
            off, tcnt = carry
            rvec = plsc.load_gather(
                blist, [jnp.full((16,), u, dtype=jnp.int32)])
            base = jnp.sum(jnp.where(lane == 0, rvec, 0)) * NC

            def ichunk(t, c2):
                off2, tc2 = c2
                v = gbuf[u, pl.ds(t * 16, 16)]
                mask = v <= tsv
                plsc.store_compressed(candv.at[pl.ds(off2, 16)], v,
                                      mask=mask)
                plsc.store_compressed(candi.at[pl.ds(off2, 16)],
                                      base + t * 16 + lane, mask=mask)
                cnt = jnp.sum(mask.astype(jnp.int32))
                return jnp.minimum(off2 + cnt, CAP - 16), tc2 + cnt

            return lax.fori_loop(0, NC // 16, ichunk, (off, tcnt))

        off, tcnt = lax.fori_loop(0, nbc, rchunk,
                                  (jnp.int32(0), jnp.int32(0)))
        candv[pl.ds(off, 16)] = inf16
        candi[pl.ds(off, 16)] = big16
        nv = off // 16 + 1

        def run_select(load_pair, nvec):
            def select(k, carry):
                mprev, iprev, sel0, sel1 = carry

                def pass1(t, mv):
                    cv, ci = load_pair(t)
                    elig = (cv > mprev) | ((cv == mprev) & (ci > iprev))
                    return jnp.minimum(mv, jnp.where(elig, cv, inf16))

                m = jnp.min(lax.fori_loop(0, nvec, pass1, inf16))

                def pass2(t, iv):
                    cv, ci = load_pair(t)
                    elig = (cv == m) & ((cv > mprev) | (ci > iprev))
                    return jnp.minimum(iv, jnp.where(elig, ci, big16))

                i = jnp.min(lax.fori_loop(0, nvec, pass2, big16))
                sel0 = jnp.where(lane == k, i, sel0)
                sel1 = jnp.where(lane == (k - 16), i, sel1)
                return m, i, sel0, sel1

            zero16 = jnp.zeros((16,), dtype=jnp.int32)
            _, _, sel0, sel1 = lax.fori_loop(
                0, M, select, (jnp.float32(-jnp.inf), jnp.int32(-1),
                               zero16, zero16))
            return sel0, sel1

        def load_cand(t):
            return candv[pl.ds(t * 16, 16)], candi[pl.ds(t * 16, 16)]

        def full_select():
            pltpu.sync_copy(d2_hbm.at[pl.ds(rowbase, NR)], dv)

            def load_full(t):
                return (dv[t >> 3, pl.ds((t & 7) * 16, 16)],
                        t * 16 + lane)

            return run_select(load_full, jnp.int32(N // 16))

        sel0, sel1 = lax.cond(
            (tcnt <= CAP - 16) & (nb <= CAPR),
            lambda: run_select(load_cand, nv),
            full_select)

        gl = jnp.full((16,), r * RPW + q, dtype=jnp.int32)
        hx = plsc.load_gather(cxv, [gl])
        hy = plsc.load_gather(cyv, [gl])
        hz = plsc.load_gather(czv, [gl])
        o = q * M
        ov[pl.ds(o, 16)] = plsc.load_gather(xv, [sel0]) - hx
        ov[pl.ds(o + 16, 16)] = plsc.load_gather(xv, [sel1]) - hx
        ov[pl.ds(EPW + o, 16)] = plsc.load_gather(yv, [sel0]) - hy
        ov[pl.ds(EPW + o + 16, 16)] = plsc.load_gather(yv, [sel1]) - hy
        ov[pl.ds(2 * EPW + o, 16)] = plsc.load_gather(zv, [sel0]) - hz
        ov[pl.ds(2 * EPW + o + 16, 16)] = plsc.load_gather(zv, [sel1]) - hz
        return _

    lax.fori_loop(0, RPW, row_body, 0)
    off_out = b * G * M + r * EPW
    pltpu.sync_copy(ov.at[pl.ds(0, EPW)],
                    out_hbm.at[pl.ds(0 * B * G * M + off_out, EPW)])
    pltpu.sync_copy(ov.at[pl.ds(EPW, EPW)],
                    out_hbm.at[pl.ds(1 * B * G * M + off_out, EPW)])
    pltpu.sync_copy(ov.at[pl.ds(2 * EPW, EPW)],
                    out_hbm.at[pl.ds(2 * B * G * M + off_out, EPW)])


def _sel_call(xt_flat, ct_flat, t_flat, d2_rows, rm_flat):
    mesh = plsc.VectorSubcoreMesh(core_axis_name="c", subcore_axis_name="s")
    kfn = pl.kernel(
        _sel_body,
        mesh=mesh,
        compiler_params=pltpu.CompilerParams(needs_layout_passes=False),
        out_type=jax.ShapeDtypeStruct((3 * B * G * M,), jnp.float32),
        scratch_types=[
            pltpu.VMEM((N,), jnp.float32),
            pltpu.VMEM((N,), jnp.float32),
            pltpu.VMEM((N,), jnp.float32),
            pltpu.VMEM((G,), jnp.float32),
            pltpu.VMEM((G,), jnp.float32),
            pltpu.VMEM((G,), jnp.float32),
            pltpu.VMEM((RPW,), jnp.float32),
            pltpu.VMEM((NR,), jnp.float32),
            pltpu.VMEM((NR + 16,), jnp.int32),
            pltpu.VMEM((CAPR,), jnp.int32),
            pltpu.VMEM((CAPR, NC), jnp.float32),
            pltpu.VMEM((NR, NC), jnp.float32),
            pltpu.VMEM((CAP,), jnp.float32),
            pltpu.VMEM((CAP,), jnp.int32),
            pltpu.VMEM((3 * EPW,), jnp.float32),
            pltpu.SemaphoreType.DMA,
        ],
    )
    return kfn(xt_flat, ct_flat, t_flat, d2_rows, rm_flat)


@jax.jit
def kernel(xyz):
    xt = jnp.transpose(xyz, (0, 2, 1)).reshape(B, 3, NR, NC)
    cs = _fps_call(xt.reshape(B, 3, N), xt)
    centers = jnp.transpose(cs, (0, 2, 1))
    d2, tthr, rmt = _d2_call(centers, xt)
    out3 = _sel_call(
        xt.reshape(B * 3 * N),
        cs.reshape(B * 3 * G),
        tthr.reshape(B * G),
        d2.reshape(B * G * NR, NC),
        rmt.reshape(B * G * NR),
    )
    neighborhood = jnp.transpose(out3.reshape(3, B, G, M), (1, 2, 3, 0))
    return neighborhood, centers


# speedup vs baseline: 1.5672x; 1.5672x over previous
import jax
import jax.numpy as jnp
from jax import lax
from jax.experimental import pallas as pl
from jax.experimental.pallas import tpu as pltpu
from jax.experimental.pallas import tpu_sc as plsc

B = 4
N = 16384
G = 512
M = 32
NR = 128
NC = 128

_BIG = 1 << 30


BPP = 2


def _fps_body(xs_ref, xt_ref, cs_ref):
    fiota = (lax.broadcasted_iota(jnp.int32, (NR, NC), 0) * NC
             + lax.broadcasted_iota(jnp.int32, (NR, NC), 1))
    dists0 = jnp.full((NR, NC), jnp.inf, dtype=jnp.float32)

    init = []
    for b in range(BPP):
        init += [xs_ref[b, 0, 0], xs_ref[b, 1, 0], xs_ref[b, 2, 0], dists0]

    def step(g, carry):
        out = []
        for b in range(BPP):
            px, py, pz, dists = carry[4 * b:4 * b + 4]
            cs_ref[b, 0, g] = px
            cs_ref[b, 1, g] = py
            cs_ref[b, 2, g] = pz
            dx = xt_ref[b, 0] - px
            dy = xt_ref[b, 1] - py
            dz = xt_ref[b, 2] - pz
            d = dx * dx + dy * dy + dz * dz
            dists = jnp.minimum(dists, d)
            m = jnp.max(dists)
            sel = jnp.where(dists == m, fiota, _BIG)
            i = jnp.min(sel)
            out += [xs_ref[b, 0, i], xs_ref[b, 1, i], xs_ref[b, 2, i],
                    dists]
        return tuple(out)

    lax.fori_loop(0, G, step, tuple(init))


def _fps_call(xflat, xt, interpret=False):
    call = pl.pallas_call(
        _fps_body,
        grid=(1,),
        in_specs=[
            pl.BlockSpec((BPP, 3, N), lambda i: (0, 0, 0),
                         memory_space=pltpu.SMEM),
            pl.BlockSpec((BPP, 3, NR, NC), lambda i: (0, 0, 0, 0)),
        ],
        out_specs=pl.BlockSpec((BPP, 3, G), lambda i: (0, 0, 0),
                               memory_space=pltpu.SMEM),
        out_shape=jax.ShapeDtypeStruct((BPP, 3, G), jnp.float32),
        interpret=interpret,
    )
    return jnp.concatenate(
        [call(xflat[p:p + BPP], xt[p:p + BPP])
         for p in range(0, B, BPP)], axis=0)


CPB = 8


def _d2_body(centers_ref, xt_ref, d2_ref, t_ref, rm_ref):
    x = xt_ref[0, 0]
    y = xt_ref[0, 1]
    z = xt_ref[0, 2]
    ident = (lax.broadcasted_iota(jnp.int32, (NR, NR), 0)
             == lax.broadcasted_iota(jnp.int32, (NR, NR), 1)
             ).astype(jnp.float32)
    neg_inf = jnp.float32(-jnp.inf)
    for c in range(CPB):
        cx = centers_ref[0, c, 0]
        cy = centers_ref[0, c, 1]
        cz = centers_ref[0, c, 2]
        dx = cx - x
        dy = cy - y
        dz = cz - z
        d2 = dx * dx + dy * dy + dz * dz
        d2_ref[c] = d2
        rm = jnp.min(d2, axis=1, keepdims=True)
        rmt = lax.dot_general(rm, ident, (((0,), (0,)), ((), ())),
                              precision=lax.Precision.HIGHEST)
        rank = jnp.sum((rmt < rm).astype(jnp.int32), axis=1, keepdims=True)
        t = jnp.max(jnp.where(rank < M, rm, neg_inf))
        t_ref[0, 0, c] = t
        rm_ref[c] = rmt


def _d2_call(centers, xt, interpret=False):
    return pl.pallas_call(
        _d2_body,
        grid=(B, G // CPB),
        in_specs=[
            pl.BlockSpec((1, CPB, 3), lambda b, j: (b, j, 0),
                         memory_space=pltpu.SMEM),
            pl.BlockSpec((1, 3, NR, NC), lambda b, j: (b, 0, 0, 0)),
        ],
        out_specs=[
            pl.BlockSpec((CPB, NR, NC), lambda b, j: (b * (G // CPB) + j, 0, 0)),
            pl.BlockSpec((1, 1, CPB), lambda b, j: (b * (G // CPB) + j, 0, 0),
                         memory_space=pltpu.SMEM),
            pl.BlockSpec((CPB, 1, NR), lambda b, j: (b * (G // CPB) + j, 0, 0)),
        ],
        out_shape=[
            jax.ShapeDtypeStruct((B * G, NR, NC), jnp.float32),
            jax.ShapeDtypeStruct((B * G // CPB, 1, CPB), jnp.float32),
            jax.ShapeDtypeStruct((B * G, 1, NR), jnp.float32),
        ],
        compiler_params=pltpu.CompilerParams(
            dimension_semantics=("parallel", "parallel")),
        interpret=interpret,
    )(centers, xt)


NW = 32
WPB = NW // B
RPW = G // WPB
EPW = RPW * M
CAP = 1024


def _sel_body(xt_hbm, ct_hbm, t_hbm, d2_hbm, rm_hbm, out_hbm,
              xv, yv, zv, cxv, cyv, czv, tv, rmv, blist, dv,
              candv, candi, ov):
    w = lax.axis_index("s") * 2 + lax.axis_index("c")
    b = w // WPB
    r = w % WPB
    pltpu.sync_copy(xt_hbm.at[pl.ds(b * 3 * N, N)], xv)
    pltpu.sync_copy(xt_hbm.at[pl.ds((b * 3 + 1) * N, N)], yv)
    pltpu.sync_copy(xt_hbm.at[pl.ds((b * 3 + 2) * N, N)], zv)
    pltpu.sync_copy(ct_hbm.at[pl.ds(b * 3 * G, G)], cxv)
    pltpu.sync_copy(ct_hbm.at[pl.ds((b * 3 + 1) * G, G)], cyv)
    pltpu.sync_copy(ct_hbm.at[pl.ds((b * 3 + 2) * G, G)], czv)
    row0 = b * G + r * RPW
    pltpu.sync_copy(t_hbm.at[pl.ds(row0, RPW)], tv)

    lane = lax.iota(jnp.int32, 16)
    inf16 = jnp.full((16,), jnp.inf, dtype=jnp.float32)
    big16 = jnp.full((16,), _BIG, dtype=jnp.int32)

    def row_body(q, _):
        pltpu.sync_copy(rm_hbm.at[pl.ds((row0 + q) * NR, NR)], rmv)
        tsv = plsc.load_gather(tv, [jnp.full((16,), q, dtype=jnp.int32)])

        def fchunk(t, off):
            rv = rmv[pl.ds(t * 16, 16)]
            mask = rv <= tsv
            plsc.store_compressed(blist.at[pl.ds(off, 16)],
                                  t * 16 + lane, mask=mask)
            return off + jnp.sum(mask.astype(jnp.int32))

        nb = lax.fori_loop(0, NR // 16, fchunk, jnp.int32(0))

        rowbase = (row0 + q) * NR
        pltpu.sync_copy(d2_hbm.at[pl.ds(rowbase, NR)], dv)

        def rchunk(u, carry):
            off, tcnt = carry
            rvec = plsc.load_gather(
                blist, [jnp.full((16,), u, dtype=jnp.int32)])
            row = jnp.sum(jnp.where(lane == 0, rvec, 0))
            base = row * NC

            def ichunk(t, c2):
                off2, tc2 = c2
                v = dv[row, pl.ds(t * 16, 16)]
                mask = v <= tsv
                plsc.store_compressed(candv.at[pl.ds(off2, 16)], v,
                                      mask=mask)
                plsc.store_compressed(candi.at[pl.ds(off2, 16)],
                                      base + t * 16 + lane, mask=mask)
                cnt = jnp.sum(mask.astype(jnp.int32))
                return jnp.minimum(off2 + cnt, CAP - 16), tc2 + cnt

            return lax.fori_loop(0, NC // 16, ichunk, (off, tcnt))

        off, tcnt = lax.fori_loop(0, nb, rchunk,
                                  (jnp.int32(0), jnp.int32(0)))
        candv[pl.ds(off, 16)] = inf16
        candi[pl.ds(off, 16)] = big16
        nv = off // 16 + 1

        def run_select(load_pair, nvec):
            def select(k, carry):
                mprev, iprev, sel0, sel1 = carry

                def pass1(t, mv):
                    cv, ci = load_pair(t)
                    elig = (cv > mprev) | ((cv == mprev) & (ci > iprev))
                    return jnp.minimum(mv, jnp.where(elig, cv, inf16))

                m = jnp.min(lax.fori_loop(0, nvec, pass1, inf16))

                def pass2(t, iv):
                    cv, ci = load_pair(t)
                    elig = (cv == m) & ((cv > mprev) | (ci > iprev))
                    return jnp.minimum(iv, jnp.where(elig, ci, big16))

                i = jnp.min(lax.fori_loop(0, nvec, pass2, big16))
                sel0 = jnp.where(lane == k, i, sel0)
                sel1 = jnp.where(lane == (k - 16), i, sel1)
                return m, i, sel0, sel1

            zero16 = jnp.zeros((16,), dtype=jnp.int32)
            _, _, sel0, sel1 = lax.fori_loop(
                0, M, select, (jnp.float32(-jnp.inf), jnp.int32(-1),
                               zero16, zero16))
            return sel0, sel1

        def load_cand(t):
            return candv[pl.ds(t * 16, 16)], candi[pl.ds(t * 16, 16)]

        def load_full(t):
            return (dv[t >> 3, pl.ds((t & 7) * 16, 16)],
                    t * 16 + lane)

        sel0, sel1 = lax.cond(
            tcnt <= CAP - 16,
            lambda: run_select(load_cand, nv),
            lambda: run_select(load_full, jnp.int32(N // 16)))

        gl = jnp.full((16,), r * RPW + q, dtype=jnp.int32)
        hx = plsc.load_gather(cxv, [gl])
        hy = plsc.load_gather(cyv, [gl])
        hz = plsc.load_gather(czv, [gl])
        o = q * M
        ov[pl.ds(o, 16)] = plsc.load_gather(xv, [sel0]) - hx
        ov[pl.ds(o + 16, 16)] = plsc.load_gather(xv, [sel1]) - hx
        ov[pl.ds(EPW + o, 16)] = plsc.load_gather(yv, [sel0]) - hy
        ov[pl.ds(EPW + o + 16, 16)] = plsc.load_gather(yv, [sel1]) - hy
        ov[pl.ds(2 * EPW + o, 16)] = plsc.load_gather(zv, [sel0]) - hz
        ov[pl.ds(2 * EPW + o + 16, 16)] = plsc.load_gather(zv, [sel1]) - hz
        return _

    lax.fori_loop(0, RPW, row_body, 0)
    off_out = b * G * M + r * EPW
    pltpu.sync_copy(ov.at[pl.ds(0, EPW)],
                    out_hbm.at[pl.ds(0 * B * G * M + off_out, EPW)])
    pltpu.sync_copy(ov.at[pl.ds(EPW, EPW)],
                    out_hbm.at[pl.ds(1 * B * G * M + off_out, EPW)])
    pltpu.sync_copy(ov.at[pl.ds(2 * EPW, EPW)],
                    out_hbm.at[pl.ds(2 * B * G * M + off_out, EPW)])


def _sel_call(xt_flat, ct_flat, t_flat, d2_rows, rm_flat):
    mesh = plsc.VectorSubcoreMesh(core_axis_name="c", subcore_axis_name="s")
    kfn = pl.kernel(
        _sel_body,
        mesh=mesh,
        compiler_params=pltpu.CompilerParams(needs_layout_passes=False),
        out_type=jax.ShapeDtypeStruct((3 * B * G * M,), jnp.float32),
        scratch_types=[
            pltpu.VMEM((N,), jnp.float32),
            pltpu.VMEM((N,), jnp.float32),
            pltpu.VMEM((N,), jnp.float32),
            pltpu.VMEM((G,), jnp.float32),
            pltpu.VMEM((G,), jnp.float32),
            pltpu.VMEM((G,), jnp.float32),
            pltpu.VMEM((RPW,), jnp.float32),
            pltpu.VMEM((NR,), jnp.float32),
            pltpu.VMEM((NR + 16,), jnp.int32),
            pltpu.VMEM((NR, NC), jnp.float32),
            pltpu.VMEM((CAP,), jnp.float32),
            pltpu.VMEM((CAP,), jnp.int32),
            pltpu.VMEM((3 * EPW,), jnp.float32),
        ],
    )
    return kfn(xt_flat, ct_flat, t_flat, d2_rows, rm_flat)


@jax.jit
def kernel(xyz):
    xt = jnp.transpose(xyz, (0, 2, 1)).reshape(B, 3, NR, NC)
    cs = _fps_call(xt.reshape(B, 3, N), xt)
    centers = jnp.transpose(cs, (0, 2, 1))
    d2, tthr, rmt = _d2_call(centers, xt)
    out3 = _sel_call(
        xt.reshape(B * 3 * N),
        cs.reshape(B * 3 * G),
        tthr.reshape(B * G),
        d2.reshape(B * G * NR, NC),
        rmt.reshape(B * G * NR),
    )
    neighborhood = jnp.transpose(out3.reshape(3, B, G, M), (1, 2, 3, 0))
    return neighborhood, centers


# speedup vs baseline: 1.6705x; 1.0659x over previous
import jax
import jax.numpy as jnp
from jax import lax
from jax.experimental import pallas as pl
from jax.experimental.pallas import tpu as pltpu
from jax.experimental.pallas import tpu_sc as plsc

B = 4
N = 16384
G = 512
M = 32
NR = 128
NC = 128

_BIG = 1 << 30


def _fps_body(xs_ref, xt_ref, cs_ref):
    fiota = (lax.broadcasted_iota(jnp.int32, (NR, NC), 0) * NC
             + lax.broadcasted_iota(jnp.int32, (NR, NC), 1))
    dists0 = jnp.full((NR, NC), jnp.inf, dtype=jnp.float32)

    init = []
    for b in range(B):
        o = b * 3 * N
        init += [xs_ref[o], xs_ref[o + N], xs_ref[o + 2 * N], dists0]

    def step(g, carry):
        out = []
        for b in range(B):
            px, py, pz, dists = carry[4 * b:4 * b + 4]
            cs_ref[b, 0, g] = px
            cs_ref[b, 1, g] = py
            cs_ref[b, 2, g] = pz
            dx = xt_ref[b, 0] - px
            dy = xt_ref[b, 1] - py
            dz = xt_ref[b, 2] - pz
            d = dx * dx + dy * dy + dz * dz
            dists = jnp.minimum(dists, d)
            m = jnp.max(dists)
            sel = jnp.where(dists == m, fiota, _BIG)
            i = jnp.min(sel)
            o = b * 3 * N
            out += [xs_ref[o + i], xs_ref[o + N + i], xs_ref[o + 2 * N + i],
                    dists]
        return tuple(out)

    lax.fori_loop(0, G, step, tuple(init))


def _fps_call(xflat, xt, interpret=False):
    return pl.pallas_call(
        _fps_body,
        grid=(1,),
        in_specs=[
            pl.BlockSpec((B * 3 * N,), lambda i: (0,),
                         memory_space=pltpu.SMEM),
            pl.BlockSpec((B, 3, NR, NC), lambda i: (0, 0, 0, 0)),
        ],
        out_specs=pl.BlockSpec((B, 3, G), lambda i: (0, 0, 0),
                               memory_space=pltpu.SMEM),
        out_shape=jax.ShapeDtypeStruct((B, 3, G), jnp.float32),
        interpret=interpret,
    )(xflat.reshape(B * 3 * N), xt)


CPB = 8


def _d2_body(centers_ref, xt_ref, d2_ref, t_ref, rm_ref):
    x = xt_ref[0, 0]
    y = xt_ref[0, 1]
    z = xt_ref[0, 2]
    ident = (lax.broadcasted_iota(jnp.int32, (NR, NR), 0)
             == lax.broadcasted_iota(jnp.int32, (NR, NR), 1)
             ).astype(jnp.float32)
    neg_inf = jnp.float32(-jnp.inf)
    for c in range(CPB):
        cx = centers_ref[0, c, 0]
        cy = centers_ref[0, c, 1]
        cz = centers_ref[0, c, 2]
        dx = cx - x
        dy = cy - y
        dz = cz - z
        d2 = dx * dx + dy * dy + dz * dz
        d2_ref[c] = d2
        rm = jnp.min(d2, axis=1, keepdims=True)
        rmt = lax.dot_general(rm, ident, (((0,), (0,)), ((), ())),
                              precision=lax.Precision.HIGHEST)
        rank = jnp.sum((rmt < rm).astype(jnp.int32), axis=1, keepdims=True)
        t = jnp.max(jnp.where(rank < M, rm, neg_inf))
        t_ref[0, 0, c] = t
        rm_ref[c] = rmt


def _d2_call(centers, xt, interpret=False):
    return pl.pallas_call(
        _d2_body,
        grid=(B, G // CPB),
        in_specs=[
            pl.BlockSpec((1, CPB, 3), lambda b, j: (b, j, 0),
                         memory_space=pltpu.SMEM),
            pl.BlockSpec((1, 3, NR, NC), lambda b, j: (b, 0, 0, 0)),
        ],
        out_specs=[
            pl.BlockSpec((CPB, NR, NC), lambda b, j: (b * (G // CPB) + j, 0, 0)),
            pl.BlockSpec((1, 1, CPB), lambda b, j: (b * (G // CPB) + j, 0, 0),
                         memory_space=pltpu.SMEM),
            pl.BlockSpec((CPB, 1, NR), lambda b, j: (b * (G // CPB) + j, 0, 0)),
        ],
        out_shape=[
            jax.ShapeDtypeStruct((B * G, NR, NC), jnp.float32),
            jax.ShapeDtypeStruct((B * G // CPB, 1, CPB), jnp.float32),
            jax.ShapeDtypeStruct((B * G, 1, NR), jnp.float32),
        ],
        compiler_params=pltpu.CompilerParams(
            dimension_semantics=("parallel", "parallel")),
        interpret=interpret,
    )(centers, xt)


NW = 32
WPB = NW // B
RPW = G // WPB
EPW = RPW * M
CAP = 1024


def _sel_body(xt_hbm, ct_hbm, t_hbm, d2_hbm, rm_hbm, out_hbm,
              xv, yv, zv, cxv, cyv, czv, tv, rmv, blist, dv,
              candv, candi, ov):
    w = lax.axis_index("s") * 2 + lax.axis_index("c")
    b = w // WPB
    r = w % WPB
    pltpu.sync_copy(xt_hbm.at[pl.ds(b * 3 * N, N)], xv)
    pltpu.sync_copy(xt_hbm.at[pl.ds((b * 3 + 1) * N, N)], yv)
    pltpu.sync_copy(xt_hbm.at[pl.ds((b * 3 + 2) * N, N)], zv)
    pltpu.sync_copy(ct_hbm.at[pl.ds(b * 3 * G, G)], cxv)
    pltpu.sync_copy(ct_hbm.at[pl.ds((b * 3 + 1) * G, G)], cyv)
    pltpu.sync_copy(ct_hbm.at[pl.ds((b * 3 + 2) * G, G)], czv)
    row0 = b * G + r * RPW
    pltpu.sync_copy(t_hbm.at[pl.ds(row0, RPW)], tv)

    lane = lax.iota(jnp.int32, 16)
    inf16 = jnp.full((16,), jnp.inf, dtype=jnp.float32)
    big16 = jnp.full((16,), _BIG, dtype=jnp.int32)

    def row_body(q, _):
        pltpu.sync_copy(rm_hbm.at[pl.ds((row0 + q) * NR, NR)], rmv)
        tsv = plsc.load_gather(tv, [jnp.full((16,), q, dtype=jnp.int32)])

        def fchunk(t, off):
            rv = rmv[pl.ds(t * 16, 16)]
            mask = rv <= tsv
            plsc.store_compressed(blist.at[pl.ds(off, 16)],
                                  t * 16 + lane, mask=mask)
            return off + jnp.sum(mask.astype(jnp.int32))

        nb = lax.fori_loop(0, NR // 16, fchunk, jnp.int32(0))

        rowbase = (row0 + q) * NR
        pltpu.sync_copy(d2_hbm.at[pl.ds(rowbase, NR)], dv)

        def rchunk(u, carry):
            off, tcnt = carry
            rvec = plsc.load_gather(
                blist, [jnp.full((16,), u, dtype=jnp.int32)])
            row = jnp.sum(jnp.where(lane == 0, rvec, 0))
            base = row * NC

            def ichunk(t, c2):
                off2, tc2 = c2
                v = dv[row, pl.ds(t * 16, 16)]
                mask = v <= tsv
                plsc.store_compressed(candv.at[pl.ds(off2, 16)], v,
                                      mask=mask)
                plsc.store_compressed(candi.at[pl.ds(off2, 16)],
                                      base + t * 16 + lane, mask=mask)
                cnt = jnp.sum(mask.astype(jnp.int32))
                return jnp.minimum(off2 + cnt, CAP - 16), tc2 + cnt

            return lax.fori_loop(0, NC // 16, ichunk, (off, tcnt))

        off, tcnt = lax.fori_loop(0, nb, rchunk,
                                  (jnp.int32(0), jnp.int32(0)))
        candv[pl.ds(off, 16)] = inf16
        candi[pl.ds(off, 16)] = big16
        nv = off // 16 + 1

        def run_select(load_pair, nvec):
            def select(k, carry):
                mprev, iprev, sel0, sel1 = carry

                def pass1(t, mv):
                    cv, ci = load_pair(t)
                    elig = (cv > mprev) | ((cv == mprev) & (ci > iprev))
                    return jnp.minimum(mv, jnp.where(elig, cv, inf16))

                m = jnp.min(lax.fori_loop(0, nvec, pass1, inf16))

                def pass2(t, iv):
                    cv, ci = load_pair(t)
                    elig = (cv == m) & ((cv > mprev) | (ci > iprev))
                    return jnp.minimum(iv, jnp.where(elig, ci, big16))

                i = jnp.min(lax.fori_loop(0, nvec, pass2, big16))
                sel0 = jnp.where(lane == k, i, sel0)
                sel1 = jnp.where(lane == (k - 16), i, sel1)
                return m, i, sel0, sel1

            zero16 = jnp.zeros((16,), dtype=jnp.int32)
            _, _, sel0, sel1 = lax.fori_loop(
                0, M, select, (jnp.float32(-jnp.inf), jnp.int32(-1),
                               zero16, zero16))
            return sel0, sel1

        def load_cand(t):
            return candv[pl.ds(t * 16, 16)], candi[pl.ds(t * 16, 16)]

        def load_full(t):
            return (dv[t >> 3, pl.ds((t & 7) * 16, 16)],
                    t * 16 + lane)

        sel0, sel1 = lax.cond(
            tcnt <= CAP - 16,
            lambda: run_select(load_cand, nv),
            lambda: run_select(load_full, jnp.int32(N // 16)))

        gl = jnp.full((16,), r * RPW + q, dtype=jnp.int32)
        hx = plsc.load_gather(cxv, [gl])
        hy = plsc.load_gather(cyv, [gl])
        hz = plsc.load_gather(czv, [gl])
        o = q * M
        ov[pl.ds(o, 16)] = plsc.load_gather(xv, [sel0]) - hx
        ov[pl.ds(o + 16, 16)] = plsc.load_gather(xv, [sel1]) - hx
        ov[pl.ds(EPW + o, 16)] = plsc.load_gather(yv, [sel0]) - hy
        ov[pl.ds(EPW + o + 16, 16)] = plsc.load_gather(yv, [sel1]) - hy
        ov[pl.ds(2 * EPW + o, 16)] = plsc.load_gather(zv, [sel0]) - hz
        ov[pl.ds(2 * EPW + o + 16, 16)] = plsc.load_gather(zv, [sel1]) - hz
        return _

    lax.fori_loop(0, RPW, row_body, 0)
    off_out = b * G * M + r * EPW
    pltpu.sync_copy(ov.at[pl.ds(0, EPW)],
                    out_hbm.at[pl.ds(0 * B * G * M + off_out, EPW)])
    pltpu.sync_copy(ov.at[pl.ds(EPW, EPW)],
                    out_hbm.at[pl.ds(1 * B * G * M + off_out, EPW)])
    pltpu.sync_copy(ov.at[pl.ds(2 * EPW, EPW)],
                    out_hbm.at[pl.ds(2 * B * G * M + off_out, EPW)])


def _sel_call(xt_flat, ct_flat, t_flat, d2_rows, rm_flat):
    mesh = plsc.VectorSubcoreMesh(core_axis_name="c", subcore_axis_name="s")
    kfn = pl.kernel(
        _sel_body,
        mesh=mesh,
        compiler_params=pltpu.CompilerParams(needs_layout_passes=False),
        out_type=jax.ShapeDtypeStruct((3 * B * G * M,), jnp.float32),
        scratch_types=[
            pltpu.VMEM((N,), jnp.float32),
            pltpu.VMEM((N,), jnp.float32),
            pltpu.VMEM((N,), jnp.float32),
            pltpu.VMEM((G,), jnp.float32),
            pltpu.VMEM((G,), jnp.float32),
            pltpu.VMEM((G,), jnp.float32),
            pltpu.VMEM((RPW,), jnp.float32),
            pltpu.VMEM((NR,), jnp.float32),
            pltpu.VMEM((NR + 16,), jnp.int32),
            pltpu.VMEM((NR, NC), jnp.float32),
            pltpu.VMEM((CAP,), jnp.float32),
            pltpu.VMEM((CAP,), jnp.int32),
            pltpu.VMEM((3 * EPW,), jnp.float32),
        ],
    )
    return kfn(xt_flat, ct_flat, t_flat, d2_rows, rm_flat)


@jax.jit
def kernel(xyz):
    xt = jnp.transpose(xyz, (0, 2, 1)).reshape(B, 3, NR, NC)
    cs = _fps_call(xt.reshape(B, 3, N), xt)
    centers = jnp.transpose(cs, (0, 2, 1))
    d2, tthr, rmt = _d2_call(centers, xt)
    out3 = _sel_call(
        xt.reshape(B * 3 * N),
        cs.reshape(B * 3 * G),
        tthr.reshape(B * G),
        d2.reshape(B * G * NR, NC),
        rmt.reshape(B * G * NR),
    )
    neighborhood = jnp.transpose(out3.reshape(3, B, G, M), (1, 2, 3, 0))
    return neighborhood, centers


# speedup vs baseline: 1.8503x; 1.1076x over previous
import jax
import jax.numpy as jnp
from jax import lax
from jax.experimental import pallas as pl
from jax.experimental.pallas import tpu as pltpu
from jax.experimental.pallas import tpu_sc as plsc

B = 4
N = 16384
G = 512
M = 32
NR = 128
NC = 128

_BIG = 1 << 30


def _fps_body(xs_ref, xt_ref, cs_ref):
    fiota = (lax.broadcasted_iota(jnp.int32, (NR, NC), 0) * NC
             + lax.broadcasted_iota(jnp.int32, (NR, NC), 1))
    dists0 = jnp.full((NR, NC), jnp.inf, dtype=jnp.float32)

    init = []
    for b in range(B):
        o = b * 3 * N
        init += [xs_ref[o], xs_ref[o + N], xs_ref[o + 2 * N], dists0]

    def step(g, carry):
        out = []
        for b in range(B):
            px, py, pz, dists = carry[4 * b:4 * b + 4]
            cs_ref[b, 0, g] = px
            cs_ref[b, 1, g] = py
            cs_ref[b, 2, g] = pz
            dx = xt_ref[b, 0] - px
            dy = xt_ref[b, 1] - py
            dz = xt_ref[b, 2] - pz
            d = dx * dx + dy * dy + dz * dz
            dists = jnp.minimum(dists, d)
            m = jnp.max(dists)
            sel = jnp.where(dists == m, fiota, _BIG)
            i = jnp.min(sel)
            o = b * 3 * N
            out += [xs_ref[o + i], xs_ref[o + N + i], xs_ref[o + 2 * N + i],
                    dists]
        return tuple(out)

    lax.fori_loop(0, G, step, tuple(init))


def _fps_call(xflat, xt, interpret=False):
    return pl.pallas_call(
        _fps_body,
        grid=(1,),
        in_specs=[
            pl.BlockSpec((B * 3 * N,), lambda i: (0,),
                         memory_space=pltpu.SMEM),
            pl.BlockSpec((B, 3, NR, NC), lambda i: (0, 0, 0, 0)),
        ],
        out_specs=pl.BlockSpec((B, 3, G), lambda i: (0, 0, 0),
                               memory_space=pltpu.SMEM),
        out_shape=jax.ShapeDtypeStruct((B, 3, G), jnp.float32),
        interpret=interpret,
    )(xflat.reshape(B * 3 * N), xt)


CPB = 8


def _d2_body(centers_ref, xt_ref, d2_ref, t_ref, rm_ref):
    x = xt_ref[0, 0]
    y = xt_ref[0, 1]
    z = xt_ref[0, 2]
    ident = (lax.broadcasted_iota(jnp.int32, (NR, NR), 0)
             == lax.broadcasted_iota(jnp.int32, (NR, NR), 1)
             ).astype(jnp.float32)
    neg_inf = jnp.float32(-jnp.inf)
    for c in range(CPB):
        cx = centers_ref[0, c, 0]
        cy = centers_ref[0, c, 1]
        cz = centers_ref[0, c, 2]
        dx = cx - x
        dy = cy - y
        dz = cz - z
        d2 = dx * dx + dy * dy + dz * dz
        d2_ref[c] = d2
        rm = jnp.min(d2, axis=1, keepdims=True)
        rmt = lax.dot_general(rm, ident, (((0,), (0,)), ((), ())),
                              precision=lax.Precision.HIGHEST)
        rank = jnp.sum((rmt < rm).astype(jnp.int32), axis=1, keepdims=True)
        t = jnp.max(jnp.where(rank < M, rm, neg_inf))
        t_ref[0, 0, c] = t
        rm_ref[c] = rmt


def _d2_call(centers, xt, interpret=False):
    return pl.pallas_call(
        _d2_body,
        grid=(PB, G // CPB),
        in_specs=[
            pl.BlockSpec((1, CPB, 3), lambda b, j: (b, j, 0),
                         memory_space=pltpu.SMEM),
            pl.BlockSpec((1, 3, NR, NC), lambda b, j: (b, 0, 0, 0)),
        ],
        out_specs=[
            pl.BlockSpec((CPB, NR, NC), lambda b, j: (b * (G // CPB) + j, 0, 0)),
            pl.BlockSpec((1, 1, CPB), lambda b, j: (b * (G // CPB) + j, 0, 0),
                         memory_space=pltpu.SMEM),
            pl.BlockSpec((CPB, 1, NR), lambda b, j: (b * (G // CPB) + j, 0, 0)),
        ],
        out_shape=[
            jax.ShapeDtypeStruct((PB * G, NR, NC), jnp.float32),
            jax.ShapeDtypeStruct((PB * G // CPB, 1, CPB), jnp.float32),
            jax.ShapeDtypeStruct((PB * G, 1, NR), jnp.float32),
        ],
        compiler_params=pltpu.CompilerParams(
            dimension_semantics=("parallel", "parallel")),
        interpret=interpret,
    )(centers, xt)


NW = 32
PB = 2
WPB = NW // PB
RPW = G // WPB
EPW = RPW * M
CAP = 1024


def _sel_body(xt_hbm, ct_hbm, t_hbm, d2_hbm, rm_hbm, out_hbm,
              xv, yv, zv, cxv, cyv, czv, tv, rmv, blist, dv,
              candv, candi, ov):
    w = lax.axis_index("s") * 2 + lax.axis_index("c")
    b = w // WPB
    r = w % WPB
    pltpu.sync_copy(xt_hbm.at[pl.ds(b * 3 * N, N)], xv)
    pltpu.sync_copy(xt_hbm.at[pl.ds((b * 3 + 1) * N, N)], yv)
    pltpu.sync_copy(xt_hbm.at[pl.ds((b * 3 + 2) * N, N)], zv)
    pltpu.sync_copy(ct_hbm.at[pl.ds(b * 3 * G, G)], cxv)
    pltpu.sync_copy(ct_hbm.at[pl.ds((b * 3 + 1) * G, G)], cyv)
    pltpu.sync_copy(ct_hbm.at[pl.ds((b * 3 + 2) * G, G)], czv)
    row0 = b * G + r * RPW
    pltpu.sync_copy(t_hbm.at[pl.ds(row0, RPW)], tv)

    lane = lax.iota(jnp.int32, 16)
    inf16 = jnp.full((16,), jnp.inf, dtype=jnp.float32)
    big16 = jnp.full((16,), _BIG, dtype=jnp.int32)

    def row_body(q, _):
        pltpu.sync_copy(rm_hbm.at[pl.ds((row0 + q) * NR, NR)], rmv)
        tsv = plsc.load_gather(tv, [jnp.full((16,), q, dtype=jnp.int32)])

        def fchunk(t, off):
            rv = rmv[pl.ds(t * 16, 16)]
            mask = rv <= tsv
            plsc.store_compressed(blist.at[pl.ds(off, 16)],
                                  t * 16 + lane, mask=mask)
            return off + jnp.sum(mask.astype(jnp.int32))

        nb = lax.fori_loop(0, NR // 16, fchunk, jnp.int32(0))

        rowbase = (row0 + q) * NR
        pltpu.sync_copy(d2_hbm.at[pl.ds(rowbase, NR)], dv)

        def rchunk(u, carry):
            off, tcnt = carry
            rvec = plsc.load_gather(
                blist, [jnp.full((16,), u, dtype=jnp.int32)])
            row = jnp.sum(jnp.where(lane == 0, rvec, 0))
            base = row * NC

            def ichunk(t, c2):
                off2, tc2 = c2
                v = dv[row, pl.ds(t * 16, 16)]
                mask = v <= tsv
                plsc.store_compressed(candv.at[pl.ds(off2, 16)], v,
                                      mask=mask)
                plsc.store_compressed(candi.at[pl.ds(off2, 16)],
                                      base + t * 16 + lane, mask=mask)
                cnt = jnp.sum(mask.astype(jnp.int32))
                return jnp.minimum(off2 + cnt, CAP - 16), tc2 + cnt

            return lax.fori_loop(0, NC // 16, ichunk, (off, tcnt))

        off, tcnt = lax.fori_loop(0, nb, rchunk,
                                  (jnp.int32(0), jnp.int32(0)))
        candv[pl.ds(off, 16)] = inf16
        candi[pl.ds(off, 16)] = big16
        nv = off // 16 + 1

        def run_select(load_pair, nvec):
            def select(k, carry):
                mprev, iprev, sel0, sel1 = carry

                def pass1(t, mv):
                    cv, ci = load_pair(t)
                    elig = (cv > mprev) | ((cv == mprev) & (ci > iprev))
                    return jnp.minimum(mv, jnp.where(elig, cv, inf16))

                m = jnp.min(lax.fori_loop(0, nvec, pass1, inf16))

                def pass2(t, iv):
                    cv, ci = load_pair(t)
                    elig = (cv == m) & ((cv > mprev) | (ci > iprev))
                    return jnp.minimum(iv, jnp.where(elig, ci, big16))

                i = jnp.min(lax.fori_loop(0, nvec, pass2, big16))
                sel0 = jnp.where(lane == k, i, sel0)
                sel1 = jnp.where(lane == (k - 16), i, sel1)
                return m, i, sel0, sel1

            zero16 = jnp.zeros((16,), dtype=jnp.int32)
            _, _, sel0, sel1 = lax.fori_loop(
                0, M, select, (jnp.float32(-jnp.inf), jnp.int32(-1),
                               zero16, zero16))
            return sel0, sel1

        def load_cand(t):
            return candv[pl.ds(t * 16, 16)], candi[pl.ds(t * 16, 16)]

        def load_full(t):
            return (dv[t >> 3, pl.ds((t & 7) * 16, 16)],
                    t * 16 + lane)

        sel0, sel1 = lax.cond(
            tcnt <= CAP - 16,
            lambda: run_select(load_cand, nv),
            lambda: run_select(load_full, jnp.int32(N // 16)))

        gl = jnp.full((16,), r * RPW + q, dtype=jnp.int32)
        hx = plsc.load_gather(cxv, [gl])
        hy = plsc.load_gather(cyv, [gl])
        hz = plsc.load_gather(czv, [gl])
        o = q * M
        ov[pl.ds(o, 16)] = plsc.load_gather(xv, [sel0]) - hx
        ov[pl.ds(o + 16, 16)] = plsc.load_gather(xv, [sel1]) - hx
        ov[pl.ds(EPW + o, 16)] = plsc.load_gather(yv, [sel0]) - hy
        ov[pl.ds(EPW + o + 16, 16)] = plsc.load_gather(yv, [sel1]) - hy
        ov[pl.ds(2 * EPW + o, 16)] = plsc.load_gather(zv, [sel0]) - hz
        ov[pl.ds(2 * EPW + o + 16, 16)] = plsc.load_gather(zv, [sel1]) - hz
        return _

    lax.fori_loop(0, RPW, row_body, 0)
    off_out = b * G * M + r * EPW
    pltpu.sync_copy(ov.at[pl.ds(0, EPW)],
                    out_hbm.at[pl.ds(0 * PB * G * M + off_out, EPW)])
    pltpu.sync_copy(ov.at[pl.ds(EPW, EPW)],
                    out_hbm.at[pl.ds(1 * PB * G * M + off_out, EPW)])
    pltpu.sync_copy(ov.at[pl.ds(2 * EPW, EPW)],
                    out_hbm.at[pl.ds(2 * PB * G * M + off_out, EPW)])


def _sel_call(xt_flat, ct_flat, t_flat, d2_rows, rm_flat):
    mesh = plsc.VectorSubcoreMesh(core_axis_name="c", subcore_axis_name="s")
    kfn = pl.kernel(
        _sel_body,
        mesh=mesh,
        compiler_params=pltpu.CompilerParams(needs_layout_passes=False),
        out_type=jax.ShapeDtypeStruct((3 * PB * G * M,), jnp.float32),
        scratch_types=[
            pltpu.VMEM((N,), jnp.float32),
            pltpu.VMEM((N,), jnp.float32),
            pltpu.VMEM((N,), jnp.float32),
            pltpu.VMEM((G,), jnp.float32),
            pltpu.VMEM((G,), jnp.float32),
            pltpu.VMEM((G,), jnp.float32),
            pltpu.VMEM((RPW,), jnp.float32),
            pltpu.VMEM((NR,), jnp.float32),
            pltpu.VMEM((NR + 16,), jnp.int32),
            pltpu.VMEM((NR, NC), jnp.float32),
            pltpu.VMEM((CAP,), jnp.float32),
            pltpu.VMEM((CAP,), jnp.int32),
            pltpu.VMEM((3 * EPW,), jnp.float32),
        ],
    )
    return kfn(xt_flat, ct_flat, t_flat, d2_rows, rm_flat)


@jax.jit
def kernel(xyz):
    xt = jnp.transpose(xyz, (0, 2, 1)).reshape(B, 3, NR, NC)
    cs = _fps_call(xt, xt)
    centers = jnp.transpose(cs, (0, 2, 1))
    outs = []
    for p in range(0, B, PB):
        d2, tthr, rmt = _d2_call(centers[p:p + PB], xt[p:p + PB])
        out3 = _sel_call(
            xt[p:p + PB].reshape(PB * 3 * N),
            cs[p:p + PB].reshape(PB * 3 * G),
            tthr.reshape(PB * G),
            d2.reshape(PB * G * NR, NC),
            rmt.reshape(PB * G * NR),
        )
        outs.append(out3.reshape(3, PB, G, M))
    neighborhood = jnp.transpose(jnp.concatenate(outs, axis=1),
                                 (1, 2, 3, 0))
    return neighborhood, centers
